# Initial kernel scaffold; baseline (speedup 1.0000x reference)
import jax
import jax.numpy as jnp
from jax.experimental import pallas as pl


def kernel(x, edge_index, edge_label_index, W1, b1, W2, b2, Wp1, bp1, Wp2, bp2, Wp3, bp3):
    raise NotImplementedError("write your pallas kernel here")


# speedup vs baseline: 14.3894x; 14.3894x over previous
import functools

import jax
import jax.numpy as jnp
from jax import lax
from jax.experimental import pallas as pl
from jax.experimental.pallas import tpu as pltpu
from jax.experimental.pallas import tpu_sc as plsc

N_NODES = 10000
DIM = 128
NC = 2
NS = 16
NW = NC * NS
CH = 128

N_PAD = 10240
ROWS_PER_TILE = N_PAD // NS
BLK = 512

_mesh = plsc.VectorSubcoreMesh(core_axis_name="c", subcore_axis_name="s")
F32 = jnp.float32


def _pad_chunk(idx, pad_vals):
    e = idx.shape[0]
    k = -(-e // (NW * CH))
    pad = k * NW * CH - e
    full = jnp.concatenate([idx, pad_vals[:pad]]) if pad else idx
    return full.reshape(NW, k, CH)


def _sc_degree(dst_idx):
    k = dst_idx.shape[1]

    @functools.partial(
        pl.kernel,
        out_type=jax.ShapeDtypeStruct((NC, N_PAD), F32),
        mesh=_mesh,
        scratch_types=[
            pltpu.VMEM((k, CH), jnp.int32),
            pltpu.VMEM((CH,), F32),
            pltpu.VMEM((ROWS_PER_TILE,), F32),
            pltpu.VMEM_SHARED((N_PAD,), F32),
        ],
    )
    def deg_kernel(dst_hbm, out_hbm, idx_v, ones_v, zrow_v, acc):
        c = lax.axis_index("c")
        s = lax.axis_index("s")
        w = s * NC + c

        @pl.loop(0, ROWS_PER_TILE // 16)
        def _(i):
            zrow_v[pl.ds(i * 16, 16)] = jnp.zeros((16,), F32)

        @pl.loop(0, CH // 16)
        def _(i):
            ones_v[pl.ds(i * 16, 16)] = jnp.ones((16,), F32)

        pltpu.sync_copy(zrow_v, acc.at[pl.ds(s * ROWS_PER_TILE, ROWS_PER_TILE)])
        plsc.subcore_barrier()
        pltpu.sync_copy(dst_hbm.at[w], idx_v)

        @pl.loop(0, k)
        def _(j):
            pltpu.sync_copy(ones_v, acc.at[idx_v.at[j]], add=True)

        plsc.subcore_barrier()
        pltpu.sync_copy(
            acc.at[pl.ds(s * ROWS_PER_TILE, ROWS_PER_TILE)],
            out_hbm.at[c, pl.ds(s * ROWS_PER_TILE, ROWS_PER_TILE)],
        )

    return deg_kernel(dst_idx)


def _sc_scatter(u, src_idx, dst_idx):
    k = src_idx.shape[1]

    @functools.partial(
        pl.kernel,
        out_type=jax.ShapeDtypeStruct((NC, N_PAD, DIM), F32),
        mesh=_mesh,
        scratch_types=[
            pltpu.VMEM((k, CH), jnp.int32),
            pltpu.VMEM((k, CH), jnp.int32),
            pltpu.VMEM((CH, DIM), F32),
            pltpu.VMEM((64, DIM), F32),
            pltpu.VMEM_SHARED((N_PAD, DIM), F32),
            pltpu.SemaphoreType.DMA,
        ],
    )
    def scat_kernel(u_hbm, src_hbm, dst_hbm, out_hbm, sidx, didx, rows, zbuf, acc, sem):
        c = lax.axis_index("c")
        s = lax.axis_index("s")
        w = s * NC + c

        @pl.loop(0, 64)
        def _(r):
            for cc in range(DIM // 16):
                zbuf[r, pl.ds(cc * 16, 16)] = jnp.zeros((16,), F32)

        @pl.loop(0, ROWS_PER_TILE // 64)
        def _(t):
            pltpu.sync_copy(zbuf, acc.at[pl.ds(s * ROWS_PER_TILE + t * 64, 64)])

        plsc.subcore_barrier()
        pltpu.sync_copy(src_hbm.at[w], sidx)
        pltpu.sync_copy(dst_hbm.at[w], didx)

        @pl.loop(0, k)
        def _(j):
            pltpu.async_copy(u_hbm.at[sidx.at[j]], rows, sem).wait()
            pltpu.sync_copy(rows, acc.at[didx.at[j]], add=True)

        plsc.subcore_barrier()
        pltpu.sync_copy(
            acc.at[pl.ds(s * ROWS_PER_TILE, ROWS_PER_TILE)],
            out_hbm.at[c, pl.ds(s * ROWS_PER_TILE, ROWS_PER_TILE)],
        )

    return scat_kernel(u, src_idx, dst_idx)


def _sc_gather_pair(z, lsrc_idx, ldst_idx):
    kl = lsrc_idx.shape[1]
    l_pad = NW * kl * CH

    @functools.partial(
        pl.kernel,
        out_type=[
            jax.ShapeDtypeStruct((l_pad, DIM), F32),
            jax.ShapeDtypeStruct((l_pad, DIM), F32),
        ],
        mesh=_mesh,
        scratch_types=[
            pltpu.VMEM((kl, CH), jnp.int32),
            pltpu.VMEM((kl, CH), jnp.int32),
            pltpu.VMEM((CH, DIM), F32),
            pltpu.VMEM((CH, DIM), F32),
            pltpu.SemaphoreType.DMA,
            pltpu.SemaphoreType.DMA,
        ],
    )
    def gat_kernel(z_hbm, ls_hbm, ld_hbm, es_hbm, ed_hbm, sidx, didx, rs, rd, sa, sb):
        c = lax.axis_index("c")
        s = lax.axis_index("s")
        w = s * NC + c
        base = w * kl * CH

        pltpu.sync_copy(ls_hbm.at[w], sidx)
        pltpu.sync_copy(ld_hbm.at[w], didx)

        @pl.loop(0, kl)
        def _(j):
            da = pltpu.async_copy(z_hbm.at[sidx.at[j]], rs, sa)
            db = pltpu.async_copy(z_hbm.at[didx.at[j]], rd, sb)
            da.wait()
            db.wait()
            pltpu.sync_copy(rs, es_hbm.at[pl.ds(base + j * CH, CH)])
            pltpu.sync_copy(rd, ed_hbm.at[pl.ds(base + j * CH, CH)])

    return gat_kernel(z, lsrc_idx, ldst_idx)


def _tc_mm1(x_pad, w1, d0, d1):
    grid = (N_PAD // BLK,)

    def body(x_ref, w_ref, d0_ref, d1_ref, u_ref, dinv_ref):
        deg = d0_ref[...] + d1_ref[...] + 1.0
        dinv = lax.rsqrt(deg)
        dinv_ref[...] = dinv
        h = jnp.dot(x_ref[...], w_ref[...], preferred_element_type=F32)
        u_ref[...] = h * dinv[:, None]

    return pl.pallas_call(
        body,
        grid=grid,
        in_specs=[
            pl.BlockSpec((BLK, DIM), lambda i: (i, 0)),
            pl.BlockSpec((DIM, DIM), lambda i: (0, 0)),
            pl.BlockSpec((BLK,), lambda i: (i,)),
            pl.BlockSpec((BLK,), lambda i: (i,)),
        ],
        out_specs=[
            pl.BlockSpec((BLK, DIM), lambda i: (i, 0)),
            pl.BlockSpec((BLK,), lambda i: (i,)),
        ],
        out_shape=[
            jax.ShapeDtypeStruct((N_PAD, DIM), F32),
            jax.ShapeDtypeStruct((N_PAD,), F32),
        ],
    )(x_pad, w1, d0, d1)


def _tc_mm2(s0, s1, u1, dinv, b1, w2):
    grid = (N_PAD // BLK,)

    def body(s0_ref, s1_ref, u_ref, dinv_ref, b_ref, w_ref, o_ref):
        dinv = dinv_ref[...]
        h = dinv[:, None] * (s0_ref[...] + s1_ref[...] + u_ref[...])
        h = jnp.maximum(h + b_ref[...][None, :], 0.0)
        o_ref[...] = jnp.dot(h, w_ref[...], preferred_element_type=F32) * dinv[:, None]

    return pl.pallas_call(
        body,
        grid=grid,
        in_specs=[
            pl.BlockSpec((BLK, DIM), lambda i: (i, 0)),
            pl.BlockSpec((BLK, DIM), lambda i: (i, 0)),
            pl.BlockSpec((BLK, DIM), lambda i: (i, 0)),
            pl.BlockSpec((BLK,), lambda i: (i,)),
            pl.BlockSpec((DIM,), lambda i: (0,)),
            pl.BlockSpec((DIM, DIM), lambda i: (0, 0)),
        ],
        out_specs=pl.BlockSpec((BLK, DIM), lambda i: (i, 0)),
        out_shape=jax.ShapeDtypeStruct((N_PAD, DIM), F32),
    )(s0, s1, u1, dinv, b1, w2)


def _tc_z(s0, s1, u2, dinv, b2):
    grid = (N_PAD // BLK,)

    def body(s0_ref, s1_ref, u_ref, dinv_ref, b_ref, o_ref):
        dinv = dinv_ref[...]
        o_ref[...] = (
            dinv[:, None] * (s0_ref[...] + s1_ref[...] + u_ref[...])
            + b_ref[...][None, :]
        )

    return pl.pallas_call(
        body,
        grid=grid,
        in_specs=[
            pl.BlockSpec((BLK, DIM), lambda i: (i, 0)),
            pl.BlockSpec((BLK, DIM), lambda i: (i, 0)),
            pl.BlockSpec((BLK, DIM), lambda i: (i, 0)),
            pl.BlockSpec((BLK,), lambda i: (i,)),
            pl.BlockSpec((DIM,), lambda i: (0,)),
        ],
        out_specs=pl.BlockSpec((BLK, DIM), lambda i: (i, 0)),
        out_shape=jax.ShapeDtypeStruct((N_PAD, DIM), F32),
    )(s0, s1, u2, dinv, b2)


def _tc_mlp(es, ed, wp1, bp1, wp2, bp2, wp3v, bp3):
    l_pad = es.shape[0]
    grid = (l_pad // BLK,)

    def body(es_ref, ed_ref, w1_ref, b1_ref, w2_ref, b2_ref, w3_ref, b3_ref, o_ref):
        e = es_ref[...] * ed_ref[...]
        a = jnp.maximum(
            jnp.dot(e, w1_ref[...], preferred_element_type=F32) + b1_ref[...][None, :],
            0.0,
        )
        a = jnp.maximum(
            jnp.dot(a, w2_ref[...], preferred_element_type=F32) + b2_ref[...][None, :],
            0.0,
        )
        o_ref[...] = jnp.sum(a * w3_ref[...][None, :], axis=1) + jnp.sum(b3_ref[...])

    return pl.pallas_call(
        body,
        grid=grid,
        in_specs=[
            pl.BlockSpec((BLK, DIM), lambda i: (i, 0)),
            pl.BlockSpec((BLK, DIM), lambda i: (i, 0)),
            pl.BlockSpec((DIM, DIM), lambda i: (0, 0)),
            pl.BlockSpec((DIM,), lambda i: (0,)),
            pl.BlockSpec((DIM, DIM), lambda i: (0, 0)),
            pl.BlockSpec((DIM,), lambda i: (0,)),
            pl.BlockSpec((DIM,), lambda i: (0,)),
            pl.BlockSpec((1,), lambda i: (0,)),
        ],
        out_specs=pl.BlockSpec((BLK,), lambda i: (i,)),
        out_shape=jax.ShapeDtypeStruct((l_pad,), F32),
    )(es, ed, wp1, bp1, wp2, bp2, wp3v, bp3)


def kernel(x, edge_index, edge_label_index, W1, b1, W2, b2, Wp1, bp1, Wp2, bp2, Wp3, bp3):
    n_trash = N_PAD - N_NODES
    x_pad = jnp.pad(x, ((0, n_trash), (0, 0)))

    e = edge_index.shape[1]
    pad_e = NW * CH * (-(-e // (NW * CH))) - e
    ar_e = jnp.arange(max(pad_e, 1), dtype=jnp.int32)
    src_c = _pad_chunk(edge_index[0], ar_e % N_NODES)
    dst_c = _pad_chunk(edge_index[1], N_NODES + ar_e % n_trash)

    l = edge_label_index.shape[1]
    pad_l = NW * CH * (-(-l // (NW * CH))) - l
    ar_l = jnp.arange(max(pad_l, 1), dtype=jnp.int32)
    lsrc_c = _pad_chunk(edge_label_index[0], ar_l % N_NODES)
    ldst_c = _pad_chunk(edge_label_index[1], (ar_l + 7) % N_NODES)

    degp = _sc_degree(dst_c)
    u1, dinv = _tc_mm1(x_pad, W1, degp[0], degp[1])
    sp1 = _sc_scatter(u1, src_c, dst_c)
    u2 = _tc_mm2(sp1[0], sp1[1], u1, dinv, b1, W2)
    sp2 = _sc_scatter(u2, src_c, dst_c)
    z = _tc_z(sp2[0], sp2[1], u2, dinv, b2)
    es, ed = _sc_gather_pair(z, lsrc_c, ldst_c)
    scores = _tc_mlp(es, ed, Wp1, bp1, Wp2, bp2, Wp3[:, 0], bp3)
    return scores[:l]


# speedup vs baseline: 17.5345x; 1.2186x over previous
import functools

import jax
import jax.numpy as jnp
from jax import lax
from jax.experimental import pallas as pl
from jax.experimental.pallas import tpu as pltpu
from jax.experimental.pallas import tpu_sc as plsc

N_NODES = 10000
DIM = 128
NC = 2
NS = 16
NW = NC * NS
CH = 128

N_PAD = 10240
ROWS_PER_TILE = N_PAD // NS
BLK = 512

_mesh = plsc.VectorSubcoreMesh(core_axis_name="c", subcore_axis_name="s")
F32 = jnp.float32


def _pad_chunk(idx, pad_vals, mult=2):
    e = idx.shape[0]
    k = mult * (-(-e // (NW * CH * mult)))
    pad = k * NW * CH - e
    full = jnp.concatenate([idx, pad_vals[:pad]]) if pad else idx
    return full.reshape(NW, k, CH)


def _r16(v):
    return v.astype(jnp.bfloat16).astype(F32)


def _dot_bf16(a, b):
    return jnp.dot(a.astype(jnp.bfloat16), b.astype(jnp.bfloat16),
                   preferred_element_type=F32)


def _sc_degree(dst_idx):
    k = dst_idx.shape[1]

    @functools.partial(
        pl.kernel,
        out_type=jax.ShapeDtypeStruct((NC, N_PAD), F32),
        mesh=_mesh,
        scratch_types=[
            pltpu.VMEM((k, CH), jnp.int32),
            pltpu.VMEM((CH,), F32),
            pltpu.VMEM((ROWS_PER_TILE,), F32),
            pltpu.VMEM_SHARED((N_PAD,), F32),
        ],
    )
    def deg_kernel(dst_hbm, out_hbm, idx_v, ones_v, zrow_v, acc):
        c = lax.axis_index("c")
        s = lax.axis_index("s")
        w = s * NC + c

        @pl.loop(0, ROWS_PER_TILE // 16)
        def _(i):
            zrow_v[pl.ds(i * 16, 16)] = jnp.zeros((16,), F32)

        @pl.loop(0, CH // 16)
        def _(i):
            ones_v[pl.ds(i * 16, 16)] = jnp.ones((16,), F32)

        pltpu.sync_copy(zrow_v, acc.at[pl.ds(s * ROWS_PER_TILE, ROWS_PER_TILE)])
        plsc.subcore_barrier()
        pltpu.sync_copy(dst_hbm.at[w], idx_v)

        @pl.loop(0, k)
        def _(j):
            pltpu.sync_copy(ones_v, acc.at[idx_v.at[j]], add=True)

        plsc.subcore_barrier()
        pltpu.sync_copy(
            acc.at[pl.ds(s * ROWS_PER_TILE, ROWS_PER_TILE)],
            out_hbm.at[c, pl.ds(s * ROWS_PER_TILE, ROWS_PER_TILE)],
        )

    return deg_kernel(dst_idx)


def _sc_scatter(u, src_idx, dst_idx):
    k = src_idx.shape[1]
    assert k % 4 == 0

    @functools.partial(
        pl.kernel,
        out_type=jax.ShapeDtypeStruct((NC, N_PAD, DIM), F32),
        mesh=_mesh,
        scratch_types=[
            pltpu.VMEM((4, CH), jnp.int32),
            pltpu.VMEM((4, CH), jnp.int32),
            pltpu.VMEM((CH, DIM), F32),
            pltpu.VMEM((CH, DIM), F32),
            pltpu.VMEM_SHARED((N_PAD, DIM), F32),
            [pltpu.SemaphoreType.DMA] * 4,
            [pltpu.SemaphoreType.DMA] * 4,
            pltpu.SemaphoreType.DMA,
            pltpu.SemaphoreType.DMA,
        ],
    )
    def scat_kernel(u_hbm, src_hbm, dst_hbm, out_hbm, sring, dring, rows0, rows1,
                    acc, ssems, dsems, gsem0, gsem1):
        c = lax.axis_index("c")
        s = lax.axis_index("s")
        w = s * NC + c
        rows = (rows0, rows1)
        gsems = (gsem0, gsem1)

        for t in range(4):
            pltpu.async_copy(src_hbm.at[w, t], sring.at[t], ssems[t])
            pltpu.async_copy(dst_hbm.at[w, t], dring.at[t], dsems[t])

        @pl.loop(0, CH)
        def _(r):
            for cc in range(DIM // 16):
                rows0[r, pl.ds(cc * 16, 16)] = jnp.zeros((16,), F32)

        @pl.loop(0, ROWS_PER_TILE // CH)
        def _(t):
            pltpu.sync_copy(rows0, acc.at[pl.ds(s * ROWS_PER_TILE + t * CH, CH)])

        plsc.subcore_barrier()

        pltpu.make_async_copy(src_hbm.at[w, 0], sring.at[0], ssems[0]).wait()
        pltpu.async_copy(u_hbm.at[sring.at[0]], rows0, gsem0)
        pltpu.make_async_copy(src_hbm.at[w, 1], sring.at[1], ssems[1]).wait()
        pltpu.async_copy(u_hbm.at[sring.at[1]], rows1, gsem1)

        @pl.loop(0, k // 4)
        def _(j4):
            j = j4 * 4
            for t in range(4):
                jj = j + t
                rb = rows[t % 2]
                gs = gsems[t % 2]
                t2 = (t + 2) % 4
                pltpu.make_async_copy(dst_hbm.at[w, jj], dring.at[t], dsems[t]).wait()
                pltpu.make_async_copy(u_hbm.at[sring.at[t]], rb, gs).wait()
                pltpu.sync_copy(rb, acc.at[dring.at[t]], add=True)

                @pl.when(jj + 2 < k)
                def _():
                    pltpu.make_async_copy(
                        src_hbm.at[w, jj + 2], sring.at[t2], ssems[t2]
                    ).wait()
                    pltpu.async_copy(u_hbm.at[sring.at[t2]], rb, gs)

                @pl.when(jj + 4 < k)
                def _():
                    pltpu.async_copy(src_hbm.at[w, jj + 4], sring.at[t], ssems[t])
                    pltpu.async_copy(dst_hbm.at[w, jj + 4], dring.at[t], dsems[t])

        plsc.subcore_barrier()
        pltpu.sync_copy(
            acc.at[pl.ds(s * ROWS_PER_TILE, ROWS_PER_TILE)],
            out_hbm.at[c, pl.ds(s * ROWS_PER_TILE, ROWS_PER_TILE)],
        )

    return scat_kernel(u, src_idx, dst_idx)


def _sc_gather_pair(z, lsrc_idx, ldst_idx):
    kl = lsrc_idx.shape[1]
    l_pad = NW * kl * CH

    @functools.partial(
        pl.kernel,
        out_type=[
            jax.ShapeDtypeStruct((l_pad, DIM), F32),
            jax.ShapeDtypeStruct((l_pad, DIM), F32),
        ],
        mesh=_mesh,
        scratch_types=[
            pltpu.VMEM((kl, CH), jnp.int32),
            pltpu.VMEM((kl, CH), jnp.int32),
            pltpu.VMEM((CH, DIM), F32),
            pltpu.VMEM((CH, DIM), F32),
            pltpu.VMEM((CH, DIM), F32),
            pltpu.VMEM((CH, DIM), F32),
            pltpu.SemaphoreType.DMA,
            [pltpu.SemaphoreType.DMA] * 4,
            [pltpu.SemaphoreType.DMA] * 4,
        ],
    )
    def gat_kernel(z_hbm, ls_hbm, ld_hbm, es_hbm, ed_hbm, sidx, didx,
                   rs0, rd0, rs1, rd1, isem, gsems, wsems):
        c = lax.axis_index("c")
        s = lax.axis_index("s")
        w = s * NC + c
        base = w * kl * CH

        di = pltpu.async_copy(ls_hbm.at[w], sidx, isem)
        dj = pltpu.async_copy(ld_hbm.at[w], didx, isem)
        di.wait()
        dj.wait()

        pltpu.async_copy(z_hbm.at[sidx.at[0]], rs0, gsems[0])
        pltpu.async_copy(z_hbm.at[didx.at[0]], rd0, gsems[1])
        pltpu.async_copy(z_hbm.at[sidx.at[1]], rs1, gsems[2])
        pltpu.async_copy(z_hbm.at[didx.at[1]], rd1, gsems[3])

        @pl.loop(0, kl // 2)
        def _(j2):
            j = j2 * 2

            pltpu.make_async_copy(z_hbm.at[sidx.at[j]], rs0, gsems[0]).wait()
            pltpu.async_copy(rs0, es_hbm.at[pl.ds(base + j * CH, CH)], wsems[0])
            pltpu.make_async_copy(z_hbm.at[didx.at[j]], rd0, gsems[1]).wait()
            pltpu.async_copy(rd0, ed_hbm.at[pl.ds(base + j * CH, CH)], wsems[1])

            pltpu.make_async_copy(z_hbm.at[sidx.at[j + 1]], rs1, gsems[2]).wait()
            pltpu.async_copy(rs1, es_hbm.at[pl.ds(base + (j + 1) * CH, CH)], wsems[2])
            pltpu.make_async_copy(z_hbm.at[didx.at[j + 1]], rd1, gsems[3]).wait()
            pltpu.async_copy(rd1, ed_hbm.at[pl.ds(base + (j + 1) * CH, CH)], wsems[3])

            pltpu.make_async_copy(rs0, es_hbm.at[pl.ds(base + j * CH, CH)], wsems[0]).wait()
            pltpu.make_async_copy(rd0, ed_hbm.at[pl.ds(base + j * CH, CH)], wsems[1]).wait()

            @pl.when(j + 2 < kl)
            def _():
                pltpu.async_copy(z_hbm.at[sidx.at[j + 2]], rs0, gsems[0])
                pltpu.async_copy(z_hbm.at[didx.at[j + 2]], rd0, gsems[1])

            pltpu.make_async_copy(rs1, es_hbm.at[pl.ds(base + (j + 1) * CH, CH)], wsems[2]).wait()
            pltpu.make_async_copy(rd1, ed_hbm.at[pl.ds(base + (j + 1) * CH, CH)], wsems[3]).wait()

            @pl.when(j + 3 < kl)
            def _():
                pltpu.async_copy(z_hbm.at[sidx.at[j + 3]], rs1, gsems[2])
                pltpu.async_copy(z_hbm.at[didx.at[j + 3]], rd1, gsems[3])

    return gat_kernel(z, lsrc_idx, ldst_idx)


def _tc_mm1(x_pad, w1, d0, d1):
    grid = (N_PAD // BLK,)

    def body(x_ref, w_ref, d0_ref, d1_ref, u_ref, dinv_ref):
        deg = d0_ref[...] + d1_ref[...] + 1.0
        dinv = lax.rsqrt(deg)
        dinv_ref[...] = dinv
        h = _dot_bf16(x_ref[...], w_ref[...])
        u_ref[...] = h * dinv[:, None]

    return pl.pallas_call(
        body,
        grid=grid,
        in_specs=[
            pl.BlockSpec((BLK, DIM), lambda i: (i, 0)),
            pl.BlockSpec((DIM, DIM), lambda i: (0, 0)),
            pl.BlockSpec((BLK,), lambda i: (i,)),
            pl.BlockSpec((BLK,), lambda i: (i,)),
        ],
        out_specs=[
            pl.BlockSpec((BLK, DIM), lambda i: (i, 0)),
            pl.BlockSpec((BLK,), lambda i: (i,)),
        ],
        out_shape=[
            jax.ShapeDtypeStruct((N_PAD, DIM), F32),
            jax.ShapeDtypeStruct((N_PAD,), F32),
        ],
    )(x_pad, w1, d0, d1)


def _tc_mm2(s0, s1, u1, dinv, b1, w2):
    grid = (N_PAD // BLK,)

    def body(s0_ref, s1_ref, u_ref, dinv_ref, b_ref, w_ref, o_ref):
        dinv = dinv_ref[...]
        h = dinv[:, None] * (s0_ref[...] + s1_ref[...] + u_ref[...])
        h = jnp.maximum(h + b_ref[...][None, :], 0.0)
        o_ref[...] = _dot_bf16(h, w_ref[...]) * dinv[:, None]

    return pl.pallas_call(
        body,
        grid=grid,
        in_specs=[
            pl.BlockSpec((BLK, DIM), lambda i: (i, 0)),
            pl.BlockSpec((BLK, DIM), lambda i: (i, 0)),
            pl.BlockSpec((BLK, DIM), lambda i: (i, 0)),
            pl.BlockSpec((BLK,), lambda i: (i,)),
            pl.BlockSpec((DIM,), lambda i: (0,)),
            pl.BlockSpec((DIM, DIM), lambda i: (0, 0)),
        ],
        out_specs=pl.BlockSpec((BLK, DIM), lambda i: (i, 0)),
        out_shape=jax.ShapeDtypeStruct((N_PAD, DIM), F32),
    )(s0, s1, u1, dinv, b1, w2)


def _tc_z(s0, s1, u2, dinv, b2):
    grid = (N_PAD // BLK,)

    def body(s0_ref, s1_ref, u_ref, dinv_ref, b_ref, o_ref):
        dinv = dinv_ref[...]
        o_ref[...] = (
            dinv[:, None] * (s0_ref[...] + s1_ref[...] + u_ref[...])
            + b_ref[...][None, :]
        )

    return pl.pallas_call(
        body,
        grid=grid,
        in_specs=[
            pl.BlockSpec((BLK, DIM), lambda i: (i, 0)),
            pl.BlockSpec((BLK, DIM), lambda i: (i, 0)),
            pl.BlockSpec((BLK, DIM), lambda i: (i, 0)),
            pl.BlockSpec((BLK,), lambda i: (i,)),
            pl.BlockSpec((DIM,), lambda i: (0,)),
        ],
        out_specs=pl.BlockSpec((BLK, DIM), lambda i: (i, 0)),
        out_shape=jax.ShapeDtypeStruct((N_PAD, DIM), F32),
    )(s0, s1, u2, dinv, b2)


def _tc_mlp(es, ed, wp1, bp1, wp2, bp2, wp3v, bp3):
    l_pad = es.shape[0]
    grid = (l_pad // BLK,)

    def body(es_ref, ed_ref, w1_ref, b1_ref, w2_ref, b2_ref, w3_ref, b3_ref, o_ref):
        e = es_ref[...] * ed_ref[...]
        a = jnp.maximum(_dot_bf16(e, w1_ref[...]) + b1_ref[...][None, :], 0.0)
        a = jnp.maximum(_dot_bf16(a, w2_ref[...]) + b2_ref[...][None, :], 0.0)
        a16 = _r16(a)
        w316 = _r16(w3_ref[...])
        o_ref[...] = jnp.sum(a16 * w316[None, :], axis=1) + jnp.sum(b3_ref[...])

    return pl.pallas_call(
        body,
        grid=grid,
        in_specs=[
            pl.BlockSpec((BLK, DIM), lambda i: (i, 0)),
            pl.BlockSpec((BLK, DIM), lambda i: (i, 0)),
            pl.BlockSpec((DIM, DIM), lambda i: (0, 0)),
            pl.BlockSpec((DIM,), lambda i: (0,)),
            pl.BlockSpec((DIM, DIM), lambda i: (0, 0)),
            pl.BlockSpec((DIM,), lambda i: (0,)),
            pl.BlockSpec((DIM,), lambda i: (0,)),
            pl.BlockSpec((1,), lambda i: (0,)),
        ],
        out_specs=pl.BlockSpec((BLK,), lambda i: (i,)),
        out_shape=jax.ShapeDtypeStruct((l_pad,), F32),
    )(es, ed, wp1, bp1, wp2, bp2, wp3v, bp3)


def kernel(x, edge_index, edge_label_index, W1, b1, W2, b2, Wp1, bp1, Wp2, bp2, Wp3, bp3):
    n_trash = N_PAD - N_NODES
    x_pad = jnp.pad(x, ((0, n_trash), (0, 0)))

    e = edge_index.shape[1]
    pad_e = NW * CH * 4 * (-(-e // (NW * CH * 4))) - e
    ar_e = jnp.arange(max(pad_e, 1), dtype=jnp.int32)
    src_c = _pad_chunk(edge_index[0], ar_e % N_NODES, mult=4)
    dst_c = _pad_chunk(edge_index[1], N_NODES + ar_e % n_trash, mult=4)

    l = edge_label_index.shape[1]
    pad_l = NW * CH * 2 * (-(-l // (NW * CH * 2))) - l
    ar_l = jnp.arange(max(pad_l, 1), dtype=jnp.int32)
    lsrc_c = _pad_chunk(edge_label_index[0], ar_l % N_NODES)
    ldst_c = _pad_chunk(edge_label_index[1], (ar_l + 7) % N_NODES)

    degp = _sc_degree(dst_c)
    u1, dinv = _tc_mm1(x_pad, W1, degp[0], degp[1])
    sp1 = _sc_scatter(u1, src_c, dst_c)
    u2 = _tc_mm2(sp1[0], sp1[1], u1, dinv, b1, W2)
    sp2 = _sc_scatter(u2, src_c, dst_c)
    z = _tc_z(sp2[0], sp2[1], u2, dinv, b2)
    es, ed = _sc_gather_pair(z, lsrc_c, ldst_c)
    scores = _tc_mlp(es, ed, Wp1, bp1, Wp2, bp2, Wp3[:, 0], bp3)
    return scores[:l]


# speedup vs baseline: 17.6145x; 1.0046x over previous
import functools

import jax
import jax.numpy as jnp
from jax import lax
from jax.experimental import pallas as pl
from jax.experimental.pallas import tpu as pltpu
from jax.experimental.pallas import tpu_sc as plsc

N_NODES = 10000
DIM = 128
NC = 2
NS = 16
NW = NC * NS
CH = 128

N_PAD = 10240
ROWS_PER_TILE = N_PAD // NS
BLK = 512

_mesh = plsc.VectorSubcoreMesh(core_axis_name="c", subcore_axis_name="s")
F32 = jnp.float32


def _pad_chunk(idx, pad_vals, mult=2):
    e = idx.shape[0]
    k = mult * (-(-e // (NW * CH * mult)))
    pad = k * NW * CH - e
    full = jnp.concatenate([idx, pad_vals[:pad]]) if pad else idx
    return full.reshape(NW, k, CH)


def _r16(v):
    return v.astype(jnp.bfloat16).astype(F32)


def _dot_bf16(a, b):
    return jnp.dot(a.astype(jnp.bfloat16), b.astype(jnp.bfloat16),
                   preferred_element_type=F32)


def _sc_degree(dst_idx):
    k = dst_idx.shape[1]

    @functools.partial(
        pl.kernel,
        out_type=jax.ShapeDtypeStruct((NC, N_PAD), F32),
        mesh=_mesh,
        scratch_types=[
            pltpu.VMEM((k, CH), jnp.int32),
            pltpu.VMEM((CH,), F32),
            pltpu.VMEM((ROWS_PER_TILE,), F32),
            pltpu.VMEM_SHARED((N_PAD,), F32),
        ],
    )
    def deg_kernel(dst_hbm, out_hbm, idx_v, ones_v, zrow_v, acc):
        c = lax.axis_index("c")
        s = lax.axis_index("s")
        w = s * NC + c

        @pl.loop(0, ROWS_PER_TILE // 16)
        def _(i):
            zrow_v[pl.ds(i * 16, 16)] = jnp.zeros((16,), F32)

        @pl.loop(0, CH // 16)
        def _(i):
            ones_v[pl.ds(i * 16, 16)] = jnp.ones((16,), F32)

        pltpu.sync_copy(zrow_v, acc.at[pl.ds(s * ROWS_PER_TILE, ROWS_PER_TILE)])
        plsc.subcore_barrier()
        pltpu.sync_copy(dst_hbm.at[w], idx_v)

        @pl.loop(0, k)
        def _(j):
            pltpu.sync_copy(ones_v, acc.at[idx_v.at[j]], add=True)

        plsc.subcore_barrier()
        pltpu.sync_copy(
            acc.at[pl.ds(s * ROWS_PER_TILE, ROWS_PER_TILE)],
            out_hbm.at[c, pl.ds(s * ROWS_PER_TILE, ROWS_PER_TILE)],
        )

    return deg_kernel(dst_idx)


def _sc_scatter(u, src_idx, dst_idx):
    k = src_idx.shape[1]
    assert k % 4 == 0

    @functools.partial(
        pl.kernel,
        out_type=jax.ShapeDtypeStruct((NC, N_PAD, DIM), F32),
        mesh=_mesh,
        scratch_types=[
            pltpu.VMEM((4, CH), jnp.int32),
            pltpu.VMEM((4, CH), jnp.int32),
            pltpu.VMEM((CH, DIM), F32),
            pltpu.VMEM((CH, DIM), F32),
            pltpu.VMEM_SHARED((N_PAD, DIM), F32),
            [pltpu.SemaphoreType.DMA] * 4,
            [pltpu.SemaphoreType.DMA] * 4,
            pltpu.SemaphoreType.DMA,
            pltpu.SemaphoreType.DMA,
        ],
    )
    def scat_kernel(u_hbm, src_hbm, dst_hbm, out_hbm, sring, dring, rows0, rows1,
                    acc, ssems, dsems, gsem0, gsem1):
        c = lax.axis_index("c")
        s = lax.axis_index("s")
        w = s * NC + c
        rows = (rows0, rows1)
        gsems = (gsem0, gsem1)

        for t in range(4):
            pltpu.async_copy(src_hbm.at[w, t], sring.at[t], ssems[t])
            pltpu.async_copy(dst_hbm.at[w, t], dring.at[t], dsems[t])

        @pl.loop(0, CH)
        def _(r):
            for cc in range(DIM // 16):
                rows0[r, pl.ds(cc * 16, 16)] = jnp.zeros((16,), F32)

        @pl.loop(0, ROWS_PER_TILE // CH)
        def _(t):
            pltpu.sync_copy(rows0, acc.at[pl.ds(s * ROWS_PER_TILE + t * CH, CH)])

        plsc.subcore_barrier()

        pltpu.make_async_copy(src_hbm.at[w, 0], sring.at[0], ssems[0]).wait()
        pltpu.async_copy(u_hbm.at[sring.at[0]], rows0, gsem0)
        pltpu.make_async_copy(src_hbm.at[w, 1], sring.at[1], ssems[1]).wait()
        pltpu.async_copy(u_hbm.at[sring.at[1]], rows1, gsem1)

        @pl.loop(0, k // 4)
        def _(j4):
            j = j4 * 4
            for t in range(4):
                jj = j + t
                rb = rows[t % 2]
                gs = gsems[t % 2]
                t2 = (t + 2) % 4
                pltpu.make_async_copy(dst_hbm.at[w, jj], dring.at[t], dsems[t]).wait()
                pltpu.make_async_copy(u_hbm.at[sring.at[t]], rb, gs).wait()
                pltpu.sync_copy(rb, acc.at[dring.at[t]], add=True)

                @pl.when(jj + 2 < k)
                def _():
                    pltpu.make_async_copy(
                        src_hbm.at[w, jj + 2], sring.at[t2], ssems[t2]
                    ).wait()
                    pltpu.async_copy(u_hbm.at[sring.at[t2]], rb, gs)

                @pl.when(jj + 4 < k)
                def _():
                    pltpu.async_copy(src_hbm.at[w, jj + 4], sring.at[t], ssems[t])
                    pltpu.async_copy(dst_hbm.at[w, jj + 4], dring.at[t], dsems[t])

        plsc.subcore_barrier()
        pltpu.sync_copy(
            acc.at[pl.ds(s * ROWS_PER_TILE, ROWS_PER_TILE)],
            out_hbm.at[c, pl.ds(s * ROWS_PER_TILE, ROWS_PER_TILE)],
        )

    return scat_kernel(u, src_idx, dst_idx)


def _sc_gather_pair(z, lsrc_idx, ldst_idx):
    kl = lsrc_idx.shape[1]
    l_pad = NW * kl * CH

    @functools.partial(
        pl.kernel,
        out_type=[
            jax.ShapeDtypeStruct((l_pad, DIM), F32),
            jax.ShapeDtypeStruct((l_pad, DIM), F32),
        ],
        mesh=_mesh,
        scratch_types=[
            pltpu.VMEM((kl, CH), jnp.int32),
            pltpu.VMEM((kl, CH), jnp.int32),
            pltpu.VMEM((CH, DIM), F32),
            pltpu.VMEM((CH, DIM), F32),
            pltpu.VMEM((CH, DIM), F32),
            pltpu.VMEM((CH, DIM), F32),
            pltpu.SemaphoreType.DMA,
            [pltpu.SemaphoreType.DMA] * 4,
            [pltpu.SemaphoreType.DMA] * 4,
        ],
    )
    def gat_kernel(z_hbm, ls_hbm, ld_hbm, es_hbm, ed_hbm, sidx, didx,
                   rs0, rd0, rs1, rd1, isem, gsems, wsems):
        c = lax.axis_index("c")
        s = lax.axis_index("s")
        w = s * NC + c
        base = w * kl * CH

        di = pltpu.async_copy(ls_hbm.at[w], sidx, isem)
        dj = pltpu.async_copy(ld_hbm.at[w], didx, isem)
        di.wait()
        dj.wait()

        pltpu.async_copy(z_hbm.at[sidx.at[0]], rs0, gsems[0])
        pltpu.async_copy(z_hbm.at[didx.at[0]], rd0, gsems[1])
        pltpu.async_copy(z_hbm.at[sidx.at[1]], rs1, gsems[2])
        pltpu.async_copy(z_hbm.at[didx.at[1]], rd1, gsems[3])

        @pl.loop(0, kl // 2)
        def _(j2):
            j = j2 * 2

            pltpu.make_async_copy(z_hbm.at[sidx.at[j]], rs0, gsems[0]).wait()
            pltpu.async_copy(rs0, es_hbm.at[pl.ds(base + j * CH, CH)], wsems[0])
            pltpu.make_async_copy(z_hbm.at[didx.at[j]], rd0, gsems[1]).wait()
            pltpu.async_copy(rd0, ed_hbm.at[pl.ds(base + j * CH, CH)], wsems[1])

            pltpu.make_async_copy(z_hbm.at[sidx.at[j + 1]], rs1, gsems[2]).wait()
            pltpu.async_copy(rs1, es_hbm.at[pl.ds(base + (j + 1) * CH, CH)], wsems[2])
            pltpu.make_async_copy(z_hbm.at[didx.at[j + 1]], rd1, gsems[3]).wait()
            pltpu.async_copy(rd1, ed_hbm.at[pl.ds(base + (j + 1) * CH, CH)], wsems[3])

            pltpu.make_async_copy(rs0, es_hbm.at[pl.ds(base + j * CH, CH)], wsems[0]).wait()
            pltpu.make_async_copy(rd0, ed_hbm.at[pl.ds(base + j * CH, CH)], wsems[1]).wait()

            @pl.when(j + 2 < kl)
            def _():
                pltpu.async_copy(z_hbm.at[sidx.at[j + 2]], rs0, gsems[0])
                pltpu.async_copy(z_hbm.at[didx.at[j + 2]], rd0, gsems[1])

            pltpu.make_async_copy(rs1, es_hbm.at[pl.ds(base + (j + 1) * CH, CH)], wsems[2]).wait()
            pltpu.make_async_copy(rd1, ed_hbm.at[pl.ds(base + (j + 1) * CH, CH)], wsems[3]).wait()

            @pl.when(j + 3 < kl)
            def _():
                pltpu.async_copy(z_hbm.at[sidx.at[j + 3]], rs1, gsems[2])
                pltpu.async_copy(z_hbm.at[didx.at[j + 3]], rd1, gsems[3])

    return gat_kernel(z, lsrc_idx, ldst_idx)


def _tc_mm1(x_pad, w1, d0, d1):
    grid = (N_PAD // BLK,)

    def body(x_ref, w_ref, d0_ref, d1_ref, u_ref, dinv_ref):
        deg = d0_ref[...] + d1_ref[...] + 1.0
        dinv = lax.rsqrt(deg)
        dinv_ref[...] = dinv
        h = _dot_bf16(x_ref[...], w_ref[...])
        u_ref[...] = h * dinv[:, None]

    return pl.pallas_call(
        body,
        grid=grid,
        in_specs=[
            pl.BlockSpec((BLK, DIM), lambda i: (i, 0)),
            pl.BlockSpec((DIM, DIM), lambda i: (0, 0)),
            pl.BlockSpec((BLK,), lambda i: (i,)),
            pl.BlockSpec((BLK,), lambda i: (i,)),
        ],
        out_specs=[
            pl.BlockSpec((BLK, DIM), lambda i: (i, 0)),
            pl.BlockSpec((BLK,), lambda i: (i,)),
        ],
        out_shape=[
            jax.ShapeDtypeStruct((N_PAD, DIM), F32),
            jax.ShapeDtypeStruct((N_PAD,), F32),
        ],
    )(x_pad, w1, d0, d1)


def _tc_mm2(sp, u1, dinv, b1, w2):
    grid = (N_PAD // BLK,)

    def body(s0_ref, s1_ref, u_ref, dinv_ref, b_ref, w_ref, o_ref):
        dinv = dinv_ref[...]
        h = dinv[:, None] * (s0_ref[0] + s1_ref[1] + u_ref[...])
        h = jnp.maximum(h + b_ref[...][None, :], 0.0)
        o_ref[...] = _dot_bf16(h, w_ref[...]) * dinv[:, None]

    return pl.pallas_call(
        body,
        grid=grid,
        in_specs=[
            pl.BlockSpec((1, BLK, DIM), lambda i: (0, i, 0)),
            pl.BlockSpec((1, BLK, DIM), lambda i: (1, i, 0)),
            pl.BlockSpec((BLK, DIM), lambda i: (i, 0)),
            pl.BlockSpec((BLK,), lambda i: (i,)),
            pl.BlockSpec((DIM,), lambda i: (0,)),
            pl.BlockSpec((DIM, DIM), lambda i: (0, 0)),
        ],
        out_specs=pl.BlockSpec((BLK, DIM), lambda i: (i, 0)),
        out_shape=jax.ShapeDtypeStruct((N_PAD, DIM), F32),
    )(sp, sp, u1, dinv, b1, w2)


def _tc_z(sp, u2, dinv, b2):
    grid = (N_PAD // BLK,)

    def body(s0_ref, s1_ref, u_ref, dinv_ref, b_ref, o_ref):
        dinv = dinv_ref[...]
        o_ref[...] = (
            dinv[:, None] * (s0_ref[0] + s1_ref[1] + u_ref[...])
            + b_ref[...][None, :]
        )

    return pl.pallas_call(
        body,
        grid=grid,
        in_specs=[
            pl.BlockSpec((1, BLK, DIM), lambda i: (0, i, 0)),
            pl.BlockSpec((1, BLK, DIM), lambda i: (1, i, 0)),
            pl.BlockSpec((BLK, DIM), lambda i: (i, 0)),
            pl.BlockSpec((BLK,), lambda i: (i,)),
            pl.BlockSpec((DIM,), lambda i: (0,)),
        ],
        out_specs=pl.BlockSpec((BLK, DIM), lambda i: (i, 0)),
        out_shape=jax.ShapeDtypeStruct((N_PAD, DIM), F32),
    )(sp, sp, u2, dinv, b2)


def _tc_mlp(es, ed, wp1, bp1, wp2, bp2, wp3pad, bp3):
    l_pad = es.shape[0]
    grid = (l_pad // BLK,)

    def body(es_ref, ed_ref, w1_ref, b1_ref, w2_ref, b2_ref, w3_ref, b3_ref, o_ref):
        e = es_ref[...] * ed_ref[...]
        a = jnp.maximum(_dot_bf16(e, w1_ref[...]) + b1_ref[...][None, :], 0.0)
        a = jnp.maximum(_dot_bf16(a, w2_ref[...]) + b2_ref[...][None, :], 0.0)
        o_ref[...] = _dot_bf16(a, w3_ref[...]) + jnp.sum(b3_ref[...])

    return pl.pallas_call(
        body,
        grid=grid,
        in_specs=[
            pl.BlockSpec((BLK, DIM), lambda i: (i, 0)),
            pl.BlockSpec((BLK, DIM), lambda i: (i, 0)),
            pl.BlockSpec((DIM, DIM), lambda i: (0, 0)),
            pl.BlockSpec((DIM,), lambda i: (0,)),
            pl.BlockSpec((DIM, DIM), lambda i: (0, 0)),
            pl.BlockSpec((DIM,), lambda i: (0,)),
            pl.BlockSpec((DIM, 8), lambda i: (0, 0)),
            pl.BlockSpec((1,), lambda i: (0,)),
        ],
        out_specs=pl.BlockSpec((BLK, 8), lambda i: (i, 0)),
        out_shape=jax.ShapeDtypeStruct((l_pad, 8), F32),
    )(es, ed, wp1, bp1, wp2, bp2, wp3pad, bp3)


def kernel(x, edge_index, edge_label_index, W1, b1, W2, b2, Wp1, bp1, Wp2, bp2, Wp3, bp3):
    n_trash = N_PAD - N_NODES
    x_pad = jnp.pad(x, ((0, n_trash), (0, 0)))

    e = edge_index.shape[1]
    pad_e = NW * CH * 4 * (-(-e // (NW * CH * 4))) - e
    ar_e = jnp.arange(max(pad_e, 1), dtype=jnp.int32)
    src_c = _pad_chunk(edge_index[0], ar_e % N_NODES, mult=4)
    dst_c = _pad_chunk(edge_index[1], N_NODES + ar_e % n_trash, mult=4)

    l = edge_label_index.shape[1]
    pad_l = NW * CH * 2 * (-(-l // (NW * CH * 2))) - l
    ar_l = jnp.arange(max(pad_l, 1), dtype=jnp.int32)
    lsrc_c = _pad_chunk(edge_label_index[0], ar_l % N_NODES)
    ldst_c = _pad_chunk(edge_label_index[1], (ar_l + 7) % N_NODES)

    degp = _sc_degree(dst_c)
    u1, dinv = _tc_mm1(x_pad, W1, degp[0], degp[1])
    sp1 = _sc_scatter(u1, src_c, dst_c)
    u2 = _tc_mm2(sp1, u1, dinv, b1, W2)
    sp2 = _sc_scatter(u2, src_c, dst_c)
    z = _tc_z(sp2, u2, dinv, b2)
    es, ed = _sc_gather_pair(z, lsrc_c, ldst_c)
    wp3pad = jnp.pad(Wp3, ((0, 0), (0, 7)))
    scores = _tc_mlp(es, ed, Wp1, bp1, Wp2, bp2, wp3pad, bp3)
    return scores[:l, 0]


# speedup vs baseline: 17.6810x; 1.0038x over previous
import functools

import jax
import jax.numpy as jnp
from jax import lax
from jax.experimental import pallas as pl
from jax.experimental.pallas import tpu as pltpu
from jax.experimental.pallas import tpu_sc as plsc

N_NODES = 10000
DIM = 128
NC = 2
NS = 16
NW = NC * NS
CH = 128

N_PAD = 10240
ROWS_PER_TILE = N_PAD // NS
BLK = 512

_mesh = plsc.VectorSubcoreMesh(core_axis_name="c", subcore_axis_name="s")
F32 = jnp.float32


def _pad_chunk(idx, pad_vals, mult=2):
    e = idx.shape[0]
    k = mult * (-(-e // (NW * CH * mult)))
    pad = k * NW * CH - e
    full = jnp.concatenate([idx, pad_vals[:pad]]) if pad else idx
    return full.reshape(NW, k, CH)


def _r16(v):
    return v.astype(jnp.bfloat16).astype(F32)


def _dot_bf16(a, b):
    return jnp.dot(a.astype(jnp.bfloat16), b.astype(jnp.bfloat16),
                   preferred_element_type=F32)


def _sc_degree(dst_idx):
    k = dst_idx.shape[1]

    @functools.partial(
        pl.kernel,
        out_type=jax.ShapeDtypeStruct((NC, N_PAD), F32),
        mesh=_mesh,
        scratch_types=[
            pltpu.VMEM((k, CH), jnp.int32),
            pltpu.VMEM((CH,), F32),
            pltpu.VMEM((ROWS_PER_TILE,), F32),
            pltpu.VMEM_SHARED((N_PAD,), F32),
        ],
    )
    def deg_kernel(dst_hbm, out_hbm, idx_v, ones_v, zrow_v, acc):
        c = lax.axis_index("c")
        s = lax.axis_index("s")
        w = s * NC + c

        @pl.loop(0, ROWS_PER_TILE // 16)
        def _(i):
            zrow_v[pl.ds(i * 16, 16)] = jnp.zeros((16,), F32)

        @pl.loop(0, CH // 16)
        def _(i):
            ones_v[pl.ds(i * 16, 16)] = jnp.ones((16,), F32)

        pltpu.sync_copy(zrow_v, acc.at[pl.ds(s * ROWS_PER_TILE, ROWS_PER_TILE)])
        plsc.subcore_barrier()
        pltpu.sync_copy(dst_hbm.at[w], idx_v)

        @pl.loop(0, k)
        def _(j):
            pltpu.sync_copy(ones_v, acc.at[idx_v.at[j]], add=True)

        plsc.subcore_barrier()
        pltpu.sync_copy(
            acc.at[pl.ds(s * ROWS_PER_TILE, ROWS_PER_TILE)],
            out_hbm.at[c, pl.ds(s * ROWS_PER_TILE, ROWS_PER_TILE)],
        )

    return deg_kernel(dst_idx)


def _sc_scatter(u, src_idx, dst_idx):
    k = src_idx.shape[1]
    assert k % 4 == 0

    @functools.partial(
        pl.kernel,
        out_type=jax.ShapeDtypeStruct((NC, N_PAD, DIM), F32),
        mesh=_mesh,
        scratch_types=[
            pltpu.VMEM((4, CH), jnp.int32),
            pltpu.VMEM((4, CH), jnp.int32),
            pltpu.VMEM((CH, DIM), F32),
            pltpu.VMEM((CH, DIM), F32),
            pltpu.VMEM_SHARED((N_PAD, DIM), F32),
            [pltpu.SemaphoreType.DMA] * 4,
            [pltpu.SemaphoreType.DMA] * 4,
            pltpu.SemaphoreType.DMA,
            pltpu.SemaphoreType.DMA,
        ],
    )
    def scat_kernel(u_hbm, src_hbm, dst_hbm, out_hbm, sring, dring, rows0, rows1,
                    acc, ssems, dsems, gsem0, gsem1):
        c = lax.axis_index("c")
        s = lax.axis_index("s")
        w = s * NC + c
        rows = (rows0, rows1)
        gsems = (gsem0, gsem1)

        for t in range(4):
            pltpu.async_copy(src_hbm.at[w, t], sring.at[t], ssems[t])
            pltpu.async_copy(dst_hbm.at[w, t], dring.at[t], dsems[t])

        @pl.loop(0, CH)
        def _(r):
            for cc in range(DIM // 16):
                rows0[r, pl.ds(cc * 16, 16)] = jnp.zeros((16,), F32)

        @pl.loop(0, ROWS_PER_TILE // CH)
        def _(t):
            pltpu.sync_copy(rows0, acc.at[pl.ds(s * ROWS_PER_TILE + t * CH, CH)])

        plsc.subcore_barrier()

        pltpu.make_async_copy(src_hbm.at[w, 0], sring.at[0], ssems[0]).wait()
        pltpu.async_copy(u_hbm.at[sring.at[0]], rows0, gsem0)
        pltpu.make_async_copy(src_hbm.at[w, 1], sring.at[1], ssems[1]).wait()
        pltpu.async_copy(u_hbm.at[sring.at[1]], rows1, gsem1)

        @pl.loop(0, k // 4)
        def _(j4):
            j = j4 * 4
            for t in range(4):
                jj = j + t
                rb = rows[t % 2]
                gs = gsems[t % 2]
                t2 = (t + 2) % 4
                pltpu.make_async_copy(dst_hbm.at[w, jj], dring.at[t], dsems[t]).wait()
                pltpu.make_async_copy(u_hbm.at[sring.at[t]], rb, gs).wait()
                pltpu.sync_copy(rb, acc.at[dring.at[t]], add=True)

                @pl.when(jj + 2 < k)
                def _():
                    pltpu.make_async_copy(
                        src_hbm.at[w, jj + 2], sring.at[t2], ssems[t2]
                    ).wait()
                    pltpu.async_copy(u_hbm.at[sring.at[t2]], rb, gs)

                @pl.when(jj + 4 < k)
                def _():
                    pltpu.async_copy(src_hbm.at[w, jj + 4], sring.at[t], ssems[t])
                    pltpu.async_copy(dst_hbm.at[w, jj + 4], dring.at[t], dsems[t])

        plsc.subcore_barrier()
        pltpu.sync_copy(
            acc.at[pl.ds(s * ROWS_PER_TILE, ROWS_PER_TILE)],
            out_hbm.at[c, pl.ds(s * ROWS_PER_TILE, ROWS_PER_TILE)],
        )

    return scat_kernel(u, src_idx, dst_idx)


def _sc_gather_pair(z, lsrc_idx, ldst_idx):
    kl = lsrc_idx.shape[1]
    l_pad = NW * kl * CH

    @functools.partial(
        pl.kernel,
        out_type=[
            jax.ShapeDtypeStruct((l_pad, DIM), F32),
            jax.ShapeDtypeStruct((l_pad, DIM), F32),
        ],
        mesh=_mesh,
        scratch_types=[
            pltpu.VMEM((kl, CH), jnp.int32),
            pltpu.VMEM((kl, CH), jnp.int32),
            pltpu.VMEM((CH, DIM), F32),
            pltpu.VMEM((CH, DIM), F32),
            pltpu.VMEM((CH, DIM), F32),
            pltpu.VMEM((CH, DIM), F32),
            pltpu.SemaphoreType.DMA,
            [pltpu.SemaphoreType.DMA] * 4,
            [pltpu.SemaphoreType.DMA] * 4,
        ],
    )
    def gat_kernel(z_hbm, ls_hbm, ld_hbm, es_hbm, ed_hbm, sidx, didx,
                   rs0, rd0, rs1, rd1, isem, gsems, wsems):
        c = lax.axis_index("c")
        s = lax.axis_index("s")
        w = s * NC + c
        base = w * kl * CH

        di = pltpu.async_copy(ls_hbm.at[w], sidx, isem)
        dj = pltpu.async_copy(ld_hbm.at[w], didx, isem)
        di.wait()
        dj.wait()

        pltpu.async_copy(z_hbm.at[sidx.at[0]], rs0, gsems[0])
        pltpu.async_copy(z_hbm.at[didx.at[0]], rd0, gsems[1])
        pltpu.async_copy(z_hbm.at[sidx.at[1]], rs1, gsems[2])
        pltpu.async_copy(z_hbm.at[didx.at[1]], rd1, gsems[3])

        @pl.loop(0, kl // 2)
        def _(j2):
            j = j2 * 2

            pltpu.make_async_copy(z_hbm.at[sidx.at[j]], rs0, gsems[0]).wait()
            pltpu.async_copy(rs0, es_hbm.at[pl.ds(base + j * CH, CH)], wsems[0])
            pltpu.make_async_copy(z_hbm.at[didx.at[j]], rd0, gsems[1]).wait()
            pltpu.async_copy(rd0, ed_hbm.at[pl.ds(base + j * CH, CH)], wsems[1])

            pltpu.make_async_copy(z_hbm.at[sidx.at[j + 1]], rs1, gsems[2]).wait()
            pltpu.async_copy(rs1, es_hbm.at[pl.ds(base + (j + 1) * CH, CH)], wsems[2])
            pltpu.make_async_copy(z_hbm.at[didx.at[j + 1]], rd1, gsems[3]).wait()
            pltpu.async_copy(rd1, ed_hbm.at[pl.ds(base + (j + 1) * CH, CH)], wsems[3])

            pltpu.make_async_copy(rs0, es_hbm.at[pl.ds(base + j * CH, CH)], wsems[0]).wait()
            pltpu.make_async_copy(rd0, ed_hbm.at[pl.ds(base + j * CH, CH)], wsems[1]).wait()

            @pl.when(j + 2 < kl)
            def _():
                pltpu.async_copy(z_hbm.at[sidx.at[j + 2]], rs0, gsems[0])
                pltpu.async_copy(z_hbm.at[didx.at[j + 2]], rd0, gsems[1])

            pltpu.make_async_copy(rs1, es_hbm.at[pl.ds(base + (j + 1) * CH, CH)], wsems[2]).wait()
            pltpu.make_async_copy(rd1, ed_hbm.at[pl.ds(base + (j + 1) * CH, CH)], wsems[3]).wait()

            @pl.when(j + 3 < kl)
            def _():
                pltpu.async_copy(z_hbm.at[sidx.at[j + 3]], rs1, gsems[2])
                pltpu.async_copy(z_hbm.at[didx.at[j + 3]], rd1, gsems[3])

    return gat_kernel(z, lsrc_idx, ldst_idx)


def _tc_mm1(x_pad, w1, d0, d1):
    grid = (N_PAD // BLK,)

    def body(x_ref, w_ref, d0_ref, d1_ref, u_ref, dinv_ref):
        deg = d0_ref[...] + d1_ref[...] + 1.0
        dinv = lax.rsqrt(deg)
        dinv_ref[...] = dinv
        h = _dot_bf16(x_ref[...], w_ref[...])
        u_ref[...] = h * dinv[:, None]

    return pl.pallas_call(
        body,
        grid=grid,
        in_specs=[
            pl.BlockSpec((BLK, DIM), lambda i: (i, 0)),
            pl.BlockSpec((DIM, DIM), lambda i: (0, 0)),
            pl.BlockSpec((BLK,), lambda i: (i,)),
            pl.BlockSpec((BLK,), lambda i: (i,)),
        ],
        out_specs=[
            pl.BlockSpec((BLK, DIM), lambda i: (i, 0)),
            pl.BlockSpec((BLK,), lambda i: (i,)),
        ],
        out_shape=[
            jax.ShapeDtypeStruct((N_PAD, DIM), F32),
            jax.ShapeDtypeStruct((N_PAD,), F32),
        ],
    )(x_pad, w1, d0, d1)


def _tc_mm2(sp, u1, dinv, b1, w2):
    grid = (N_PAD // BLK,)

    def body(s0_ref, s1_ref, u_ref, dinv_ref, b_ref, w_ref, o_ref):
        dinv = dinv_ref[...]
        h = dinv[:, None] * (s0_ref[0] + s1_ref[0] + u_ref[...])
        h = jnp.maximum(h + b_ref[...][None, :], 0.0)
        o_ref[...] = _dot_bf16(h, w_ref[...]) * dinv[:, None]

    return pl.pallas_call(
        body,
        grid=grid,
        in_specs=[
            pl.BlockSpec((1, BLK, DIM), lambda i: (0, i, 0)),
            pl.BlockSpec((1, BLK, DIM), lambda i: (1, i, 0)),
            pl.BlockSpec((BLK, DIM), lambda i: (i, 0)),
            pl.BlockSpec((BLK,), lambda i: (i,)),
            pl.BlockSpec((DIM,), lambda i: (0,)),
            pl.BlockSpec((DIM, DIM), lambda i: (0, 0)),
        ],
        out_specs=pl.BlockSpec((BLK, DIM), lambda i: (i, 0)),
        out_shape=jax.ShapeDtypeStruct((N_PAD, DIM), F32),
    )(sp, sp, u1, dinv, b1, w2)


def _tc_z(sp, u2, dinv, b2):
    grid = (N_PAD // BLK,)

    def body(s0_ref, s1_ref, u_ref, dinv_ref, b_ref, o_ref):
        dinv = dinv_ref[...]
        o_ref[...] = (
            dinv[:, None] * (s0_ref[0] + s1_ref[0] + u_ref[...])
            + b_ref[...][None, :]
        )

    return pl.pallas_call(
        body,
        grid=grid,
        in_specs=[
            pl.BlockSpec((1, BLK, DIM), lambda i: (0, i, 0)),
            pl.BlockSpec((1, BLK, DIM), lambda i: (1, i, 0)),
            pl.BlockSpec((BLK, DIM), lambda i: (i, 0)),
            pl.BlockSpec((BLK,), lambda i: (i,)),
            pl.BlockSpec((DIM,), lambda i: (0,)),
        ],
        out_specs=pl.BlockSpec((BLK, DIM), lambda i: (i, 0)),
        out_shape=jax.ShapeDtypeStruct((N_PAD, DIM), F32),
    )(sp, sp, u2, dinv, b2)


def _tc_mlp(es, ed, wp1, bp1, wp2, bp2, wp3pad, bp3):
    l_pad = es.shape[0]
    grid = (l_pad // BLK,)

    def body(es_ref, ed_ref, w1_ref, b1_ref, w2_ref, b2_ref, w3_ref, b3_ref, o_ref):
        e = es_ref[...] * ed_ref[...]
        a = jnp.maximum(_dot_bf16(e, w1_ref[...]) + b1_ref[...][None, :], 0.0)
        a = jnp.maximum(_dot_bf16(a, w2_ref[...]) + b2_ref[...][None, :], 0.0)
        o_ref[...] = _dot_bf16(a, w3_ref[...]) + jnp.sum(b3_ref[...])

    return pl.pallas_call(
        body,
        grid=grid,
        in_specs=[
            pl.BlockSpec((BLK, DIM), lambda i: (i, 0)),
            pl.BlockSpec((BLK, DIM), lambda i: (i, 0)),
            pl.BlockSpec((DIM, DIM), lambda i: (0, 0)),
            pl.BlockSpec((DIM,), lambda i: (0,)),
            pl.BlockSpec((DIM, DIM), lambda i: (0, 0)),
            pl.BlockSpec((DIM,), lambda i: (0,)),
            pl.BlockSpec((DIM, 8), lambda i: (0, 0)),
            pl.BlockSpec((1,), lambda i: (0,)),
        ],
        out_specs=pl.BlockSpec((BLK, 8), lambda i: (i, 0)),
        out_shape=jax.ShapeDtypeStruct((l_pad, 8), F32),
    )(es, ed, wp1, bp1, wp2, bp2, wp3pad, bp3)


def kernel(x, edge_index, edge_label_index, W1, b1, W2, b2, Wp1, bp1, Wp2, bp2, Wp3, bp3):
    n_trash = N_PAD - N_NODES
    x_pad = jnp.pad(x, ((0, n_trash), (0, 0)))

    e = edge_index.shape[1]
    pad_e = NW * CH * 4 * (-(-e // (NW * CH * 4))) - e
    ar_e = jnp.arange(max(pad_e, 1), dtype=jnp.int32)
    src_c = _pad_chunk(edge_index[0], ar_e % N_NODES, mult=4)
    dst_c = _pad_chunk(edge_index[1], N_NODES + ar_e % n_trash, mult=4)

    l = edge_label_index.shape[1]
    pad_l = NW * CH * 2 * (-(-l // (NW * CH * 2))) - l
    ar_l = jnp.arange(max(pad_l, 1), dtype=jnp.int32)
    lsrc_c = _pad_chunk(edge_label_index[0], ar_l % N_NODES)
    ldst_c = _pad_chunk(edge_label_index[1], (ar_l + 7) % N_NODES)

    degp = _sc_degree(dst_c)
    u1, dinv = _tc_mm1(x_pad, W1, degp[0], degp[1])
    sp1 = _sc_scatter(u1, src_c, dst_c)
    u2 = _tc_mm2(sp1, u1, dinv, b1, W2)
    sp2 = _sc_scatter(u2, src_c, dst_c)
    z = _tc_z(sp2, u2, dinv, b2)
    es, ed = _sc_gather_pair(z, lsrc_c, ldst_c)
    wp3pad = jnp.pad(Wp3, ((0, 0), (0, 7)))
    scores = _tc_mlp(es, ed, Wp1, bp1, Wp2, bp2, wp3pad, bp3)
    return scores[:l, 0]


# speedup vs baseline: 21.1106x; 1.1940x over previous
import functools

import jax
import jax.numpy as jnp
from jax import lax
from jax.experimental import pallas as pl
from jax.experimental.pallas import tpu as pltpu
from jax.experimental.pallas import tpu_sc as plsc

N_NODES = 10000
DIM = 128
NC = 2
NS = 16
NW = NC * NS
CH = 128

N_PAD = 10240
ROWS_PER_TILE = N_PAD // NS
BLK = 2048
LBLK = 2048

_mesh = plsc.VectorSubcoreMesh(core_axis_name="c", subcore_axis_name="s")
F32 = jnp.float32


def _pad_chunk(idx, pad_vals, mult=2):
    e = idx.shape[0]
    k = mult * (-(-e // (NW * CH * mult)))
    pad = k * NW * CH - e
    full = jnp.concatenate([idx, pad_vals[:pad]]) if pad else idx
    return full.reshape(NW, k, CH)


def _r16(v):
    return v.astype(jnp.bfloat16).astype(F32)


def _dot_bf16(a, b):
    return jnp.dot(a.astype(jnp.bfloat16), b.astype(jnp.bfloat16),
                   preferred_element_type=F32)


def _sc_degree(dst_idx):
    k = dst_idx.shape[1]

    @functools.partial(
        pl.kernel,
        out_type=jax.ShapeDtypeStruct((NC, N_PAD), F32),
        mesh=_mesh,
        scratch_types=[
            pltpu.VMEM((k, CH), jnp.int32),
            pltpu.VMEM((CH,), F32),
            pltpu.VMEM((ROWS_PER_TILE,), F32),
            pltpu.VMEM_SHARED((N_PAD,), F32),
        ],
    )
    def deg_kernel(dst_hbm, out_hbm, idx_v, ones_v, zrow_v, acc):
        c = lax.axis_index("c")
        s = lax.axis_index("s")
        w = s * NC + c

        @pl.loop(0, ROWS_PER_TILE // 16)
        def _(i):
            zrow_v[pl.ds(i * 16, 16)] = jnp.zeros((16,), F32)

        @pl.loop(0, CH // 16)
        def _(i):
            ones_v[pl.ds(i * 16, 16)] = jnp.ones((16,), F32)

        pltpu.sync_copy(zrow_v, acc.at[pl.ds(s * ROWS_PER_TILE, ROWS_PER_TILE)])
        plsc.subcore_barrier()
        pltpu.sync_copy(dst_hbm.at[w], idx_v)

        @pl.loop(0, k)
        def _(j):
            pltpu.sync_copy(ones_v, acc.at[idx_v.at[j]], add=True)

        plsc.subcore_barrier()
        pltpu.sync_copy(
            acc.at[pl.ds(s * ROWS_PER_TILE, ROWS_PER_TILE)],
            out_hbm.at[c, pl.ds(s * ROWS_PER_TILE, ROWS_PER_TILE)],
        )

    return deg_kernel(dst_idx)


def _sc_scatter(u, src_idx, dst_idx):
    k = src_idx.shape[1]
    assert k % 4 == 0

    @functools.partial(
        pl.kernel,
        out_type=jax.ShapeDtypeStruct((NC, N_PAD, DIM), F32),
        mesh=_mesh,
        scratch_types=[
            pltpu.VMEM((4, CH), jnp.int32),
            pltpu.VMEM((4, CH), jnp.int32),
            pltpu.VMEM((CH, DIM), F32),
            pltpu.VMEM((CH, DIM), F32),
            pltpu.VMEM_SHARED((N_PAD, DIM), F32),
            [pltpu.SemaphoreType.DMA] * 4,
            [pltpu.SemaphoreType.DMA] * 4,
            pltpu.SemaphoreType.DMA,
            pltpu.SemaphoreType.DMA,
        ],
    )
    def scat_kernel(u_hbm, src_hbm, dst_hbm, out_hbm, sring, dring, rows0, rows1,
                    acc, ssems, dsems, gsem0, gsem1):
        c = lax.axis_index("c")
        s = lax.axis_index("s")
        w = s * NC + c
        rows = (rows0, rows1)
        gsems = (gsem0, gsem1)

        for t in range(4):
            pltpu.async_copy(src_hbm.at[w, t], sring.at[t], ssems[t])
            pltpu.async_copy(dst_hbm.at[w, t], dring.at[t], dsems[t])

        @pl.loop(0, CH)
        def _(r):
            for cc in range(DIM // 16):
                rows0[r, pl.ds(cc * 16, 16)] = jnp.zeros((16,), F32)

        @pl.loop(0, ROWS_PER_TILE // CH)
        def _(t):
            pltpu.sync_copy(rows0, acc.at[pl.ds(s * ROWS_PER_TILE + t * CH, CH)])

        plsc.subcore_barrier()

        pltpu.make_async_copy(src_hbm.at[w, 0], sring.at[0], ssems[0]).wait()
        pltpu.async_copy(u_hbm.at[sring.at[0]], rows0, gsem0)
        pltpu.make_async_copy(src_hbm.at[w, 1], sring.at[1], ssems[1]).wait()
        pltpu.async_copy(u_hbm.at[sring.at[1]], rows1, gsem1)

        @pl.loop(0, k // 4)
        def _(j4):
            j = j4 * 4
            for t in range(4):
                jj = j + t
                rb = rows[t % 2]
                gs = gsems[t % 2]
                t2 = (t + 2) % 4
                pltpu.make_async_copy(dst_hbm.at[w, jj], dring.at[t], dsems[t]).wait()
                pltpu.make_async_copy(u_hbm.at[sring.at[t]], rb, gs).wait()
                pltpu.sync_copy(rb, acc.at[dring.at[t]], add=True)

                @pl.when(jj + 2 < k)
                def _():
                    pltpu.make_async_copy(
                        src_hbm.at[w, jj + 2], sring.at[t2], ssems[t2]
                    ).wait()
                    pltpu.async_copy(u_hbm.at[sring.at[t2]], rb, gs)

                @pl.when(jj + 4 < k)
                def _():
                    pltpu.async_copy(src_hbm.at[w, jj + 4], sring.at[t], ssems[t])
                    pltpu.async_copy(dst_hbm.at[w, jj + 4], dring.at[t], dsems[t])

        plsc.subcore_barrier()
        pltpu.sync_copy(
            acc.at[pl.ds(s * ROWS_PER_TILE, ROWS_PER_TILE)],
            out_hbm.at[c, pl.ds(s * ROWS_PER_TILE, ROWS_PER_TILE)],
        )

    return scat_kernel(u, src_idx, dst_idx)


def _sc_gather_pair(z, lsrc_idx, ldst_idx):
    kl = lsrc_idx.shape[1]
    l_pad = NW * kl * CH

    @functools.partial(
        pl.kernel,
        out_type=[
            jax.ShapeDtypeStruct((l_pad, DIM), F32),
            jax.ShapeDtypeStruct((l_pad, DIM), F32),
        ],
        mesh=_mesh,
        scratch_types=[
            pltpu.VMEM((kl, CH), jnp.int32),
            pltpu.VMEM((kl, CH), jnp.int32),
            pltpu.VMEM((CH, DIM), F32),
            pltpu.VMEM((CH, DIM), F32),
            pltpu.VMEM((CH, DIM), F32),
            pltpu.VMEM((CH, DIM), F32),
            pltpu.SemaphoreType.DMA,
            [pltpu.SemaphoreType.DMA] * 4,
            [pltpu.SemaphoreType.DMA] * 4,
        ],
    )
    def gat_kernel(z_hbm, ls_hbm, ld_hbm, es_hbm, ed_hbm, sidx, didx,
                   rs0, rd0, rs1, rd1, isem, gsems, wsems):
        c = lax.axis_index("c")
        s = lax.axis_index("s")
        w = s * NC + c
        base = w * kl * CH

        di = pltpu.async_copy(ls_hbm.at[w], sidx, isem)
        dj = pltpu.async_copy(ld_hbm.at[w], didx, isem)
        di.wait()
        dj.wait()

        pltpu.async_copy(z_hbm.at[sidx.at[0]], rs0, gsems[0])
        pltpu.async_copy(z_hbm.at[didx.at[0]], rd0, gsems[1])
        pltpu.async_copy(z_hbm.at[sidx.at[1]], rs1, gsems[2])
        pltpu.async_copy(z_hbm.at[didx.at[1]], rd1, gsems[3])

        @pl.loop(0, kl // 2)
        def _(j2):
            j = j2 * 2

            pltpu.make_async_copy(z_hbm.at[sidx.at[j]], rs0, gsems[0]).wait()
            pltpu.async_copy(rs0, es_hbm.at[pl.ds(base + j * CH, CH)], wsems[0])
            pltpu.make_async_copy(z_hbm.at[didx.at[j]], rd0, gsems[1]).wait()
            pltpu.async_copy(rd0, ed_hbm.at[pl.ds(base + j * CH, CH)], wsems[1])

            pltpu.make_async_copy(z_hbm.at[sidx.at[j + 1]], rs1, gsems[2]).wait()
            pltpu.async_copy(rs1, es_hbm.at[pl.ds(base + (j + 1) * CH, CH)], wsems[2])
            pltpu.make_async_copy(z_hbm.at[didx.at[j + 1]], rd1, gsems[3]).wait()
            pltpu.async_copy(rd1, ed_hbm.at[pl.ds(base + (j + 1) * CH, CH)], wsems[3])

            pltpu.make_async_copy(rs0, es_hbm.at[pl.ds(base + j * CH, CH)], wsems[0]).wait()
            pltpu.make_async_copy(rd0, ed_hbm.at[pl.ds(base + j * CH, CH)], wsems[1]).wait()

            @pl.when(j + 2 < kl)
            def _():
                pltpu.async_copy(z_hbm.at[sidx.at[j + 2]], rs0, gsems[0])
                pltpu.async_copy(z_hbm.at[didx.at[j + 2]], rd0, gsems[1])

            pltpu.make_async_copy(rs1, es_hbm.at[pl.ds(base + (j + 1) * CH, CH)], wsems[2]).wait()
            pltpu.make_async_copy(rd1, ed_hbm.at[pl.ds(base + (j + 1) * CH, CH)], wsems[3]).wait()

            @pl.when(j + 3 < kl)
            def _():
                pltpu.async_copy(z_hbm.at[sidx.at[j + 3]], rs1, gsems[2])
                pltpu.async_copy(z_hbm.at[didx.at[j + 3]], rd1, gsems[3])

    return gat_kernel(z, lsrc_idx, ldst_idx)


def _tc_mm1(x_pad, w1, d0, d1):
    grid = (N_PAD // BLK,)

    def body(x_ref, w_ref, d0_ref, d1_ref, u_ref, dinv_ref):
        deg = d0_ref[...] + d1_ref[...] + 1.0
        dinv = lax.rsqrt(deg)
        dinv_ref[...] = dinv
        h = _dot_bf16(x_ref[...], w_ref[...])
        u_ref[...] = h * dinv[:, None]

    return pl.pallas_call(
        body,
        grid=grid,
        in_specs=[
            pl.BlockSpec((BLK, DIM), lambda i: (i, 0)),
            pl.BlockSpec((DIM, DIM), lambda i: (0, 0)),
            pl.BlockSpec((BLK,), lambda i: (i,)),
            pl.BlockSpec((BLK,), lambda i: (i,)),
        ],
        out_specs=[
            pl.BlockSpec((BLK, DIM), lambda i: (i, 0)),
            pl.BlockSpec((BLK,), lambda i: (i,)),
        ],
        out_shape=[
            jax.ShapeDtypeStruct((N_PAD, DIM), F32),
            jax.ShapeDtypeStruct((N_PAD,), F32),
        ],
    )(x_pad, w1, d0, d1)


def _tc_mm2(sp, u1, dinv, b1, w2):
    grid = (N_PAD // BLK,)

    def body(s0_ref, s1_ref, u_ref, dinv_ref, b_ref, w_ref, o_ref):
        dinv = dinv_ref[...]
        h = dinv[:, None] * (s0_ref[0] + s1_ref[0] + u_ref[...])
        h = jnp.maximum(h + b_ref[...][None, :], 0.0)
        o_ref[...] = _dot_bf16(h, w_ref[...]) * dinv[:, None]

    return pl.pallas_call(
        body,
        grid=grid,
        in_specs=[
            pl.BlockSpec((1, BLK, DIM), lambda i: (0, i, 0)),
            pl.BlockSpec((1, BLK, DIM), lambda i: (1, i, 0)),
            pl.BlockSpec((BLK, DIM), lambda i: (i, 0)),
            pl.BlockSpec((BLK,), lambda i: (i,)),
            pl.BlockSpec((DIM,), lambda i: (0,)),
            pl.BlockSpec((DIM, DIM), lambda i: (0, 0)),
        ],
        out_specs=pl.BlockSpec((BLK, DIM), lambda i: (i, 0)),
        out_shape=jax.ShapeDtypeStruct((N_PAD, DIM), F32),
    )(sp, sp, u1, dinv, b1, w2)


def _tc_z(sp, u2, dinv, b2):
    grid = (N_PAD // BLK,)

    def body(s0_ref, s1_ref, u_ref, dinv_ref, b_ref, o_ref):
        dinv = dinv_ref[...]
        o_ref[...] = (
            dinv[:, None] * (s0_ref[0] + s1_ref[0] + u_ref[...])
            + b_ref[...][None, :]
        )

    return pl.pallas_call(
        body,
        grid=grid,
        in_specs=[
            pl.BlockSpec((1, BLK, DIM), lambda i: (0, i, 0)),
            pl.BlockSpec((1, BLK, DIM), lambda i: (1, i, 0)),
            pl.BlockSpec((BLK, DIM), lambda i: (i, 0)),
            pl.BlockSpec((BLK,), lambda i: (i,)),
            pl.BlockSpec((DIM,), lambda i: (0,)),
        ],
        out_specs=pl.BlockSpec((BLK, DIM), lambda i: (i, 0)),
        out_shape=jax.ShapeDtypeStruct((N_PAD, DIM), F32),
    )(sp, sp, u2, dinv, b2)


def _tc_mlp(es, ed, wp1, bp1, wp2, bp2, wp3pad, bp3):
    l_pad = es.shape[0]
    grid = (l_pad // LBLK,)

    def body(es_ref, ed_ref, w1_ref, b1_ref, w2_ref, b2_ref, w3_ref, b3_ref, o_ref):
        e = es_ref[...] * ed_ref[...]
        a = jnp.maximum(_dot_bf16(e, w1_ref[...]) + b1_ref[...][None, :], 0.0)
        a = jnp.maximum(_dot_bf16(a, w2_ref[...]) + b2_ref[...][None, :], 0.0)
        o_ref[...] = jnp.sum(_dot_bf16(a, w3_ref[...]), axis=1) + jnp.sum(b3_ref[...])

    return pl.pallas_call(
        body,
        grid=grid,
        in_specs=[
            pl.BlockSpec((LBLK, DIM), lambda i: (i, 0)),
            pl.BlockSpec((LBLK, DIM), lambda i: (i, 0)),
            pl.BlockSpec((DIM, DIM), lambda i: (0, 0)),
            pl.BlockSpec((DIM,), lambda i: (0,)),
            pl.BlockSpec((DIM, DIM), lambda i: (0, 0)),
            pl.BlockSpec((DIM,), lambda i: (0,)),
            pl.BlockSpec((DIM, 8), lambda i: (0, 0)),
            pl.BlockSpec((1,), lambda i: (0,)),
        ],
        out_specs=pl.BlockSpec((LBLK,), lambda i: (i,)),
        out_shape=jax.ShapeDtypeStruct((l_pad,), F32),
    )(es, ed, wp1, bp1, wp2, bp2, wp3pad, bp3)


def kernel(x, edge_index, edge_label_index, W1, b1, W2, b2, Wp1, bp1, Wp2, bp2, Wp3, bp3):
    n_trash = N_PAD - N_NODES
    x_pad = jnp.pad(x, ((0, n_trash), (0, 0)))

    e = edge_index.shape[1]
    pad_e = NW * CH * 4 * (-(-e // (NW * CH * 4))) - e
    ar_e = jnp.arange(max(pad_e, 1), dtype=jnp.int32)
    src_c = _pad_chunk(edge_index[0], ar_e % N_NODES, mult=4)
    dst_c = _pad_chunk(edge_index[1], N_NODES + ar_e % n_trash, mult=4)

    l = edge_label_index.shape[1]
    pad_l = NW * CH * 2 * (-(-l // (NW * CH * 2))) - l
    ar_l = jnp.arange(max(pad_l, 1), dtype=jnp.int32)
    lsrc_c = _pad_chunk(edge_label_index[0], ar_l % N_NODES)
    ldst_c = _pad_chunk(edge_label_index[1], (ar_l + 7) % N_NODES)

    degp = _sc_degree(dst_c)
    u1, dinv = _tc_mm1(x_pad, W1, degp[0], degp[1])
    sp1 = _sc_scatter(u1, src_c, dst_c)
    u2 = _tc_mm2(sp1, u1, dinv, b1, W2)
    sp2 = _sc_scatter(u2, src_c, dst_c)
    z = _tc_z(sp2, u2, dinv, b2)
    es, ed = _sc_gather_pair(z, lsrc_c, ldst_c)
    wp3pad = jnp.pad(Wp3, ((0, 0), (0, 7)))
    scores = _tc_mlp(es, ed, Wp1, bp1, Wp2, bp2, wp3pad, bp3)
    return scores[:l]


# speedup vs baseline: 21.6938x; 1.0276x over previous
import functools

import jax
import jax.numpy as jnp
from jax import lax
from jax.experimental import pallas as pl
from jax.experimental.pallas import tpu as pltpu
from jax.experimental.pallas import tpu_sc as plsc

N_NODES = 10000
DIM = 128
NC = 2
NS = 16
NW = NC * NS
CH = 128

N_PAD = 10240
ROWS_PER_TILE = N_PAD // NS
BLK = 2048
LBLK = 2048

_mesh = plsc.VectorSubcoreMesh(core_axis_name="c", subcore_axis_name="s")
F32 = jnp.float32


def _pad_chunk(idx, pad_vals, mult=2):
    e = idx.shape[0]
    k = mult * (-(-e // (NW * CH * mult)))
    pad = k * NW * CH - e
    full = jnp.concatenate([idx, pad_vals[:pad]]) if pad else idx
    return full.reshape(NW, k, CH)


def _r16(v):
    return v.astype(jnp.bfloat16).astype(F32)


def _dot_bf16(a, b):
    return jnp.dot(a.astype(jnp.bfloat16), b.astype(jnp.bfloat16),
                   preferred_element_type=F32)


def _sc_degree(dst_idx):
    k = dst_idx.shape[1]

    @functools.partial(
        pl.kernel,
        out_type=jax.ShapeDtypeStruct((NC, N_PAD), F32),
        mesh=_mesh,
        scratch_types=[
            pltpu.VMEM((k, CH), jnp.int32),
            pltpu.VMEM((CH,), F32),
            pltpu.VMEM((ROWS_PER_TILE,), F32),
            pltpu.VMEM_SHARED((N_PAD,), F32),
        ],
    )
    def deg_kernel(dst_hbm, out_hbm, idx_v, ones_v, zrow_v, acc):
        c = lax.axis_index("c")
        s = lax.axis_index("s")
        w = s * NC + c

        @pl.loop(0, ROWS_PER_TILE // 16)
        def _(i):
            zrow_v[pl.ds(i * 16, 16)] = jnp.zeros((16,), F32)

        @pl.loop(0, CH // 16)
        def _(i):
            ones_v[pl.ds(i * 16, 16)] = jnp.ones((16,), F32)

        pltpu.sync_copy(zrow_v, acc.at[pl.ds(s * ROWS_PER_TILE, ROWS_PER_TILE)])
        plsc.subcore_barrier()
        pltpu.sync_copy(dst_hbm.at[w], idx_v)

        @pl.loop(0, k)
        def _(j):
            pltpu.sync_copy(ones_v, acc.at[idx_v.at[j]], add=True)

        plsc.subcore_barrier()
        pltpu.sync_copy(
            acc.at[pl.ds(s * ROWS_PER_TILE, ROWS_PER_TILE)],
            out_hbm.at[c, pl.ds(s * ROWS_PER_TILE, ROWS_PER_TILE)],
        )

    return deg_kernel(dst_idx)


def _sc_scatter(u, src_idx, dst_idx):
    k = src_idx.shape[1]
    assert k % 4 == 0

    @functools.partial(
        pl.kernel,
        out_type=jax.ShapeDtypeStruct((NC, N_PAD, DIM), F32),
        mesh=_mesh,
        scratch_types=[
            pltpu.VMEM((4, CH), jnp.int32),
            pltpu.VMEM((4, CH), jnp.int32),
            pltpu.VMEM((CH, DIM), F32),
            pltpu.VMEM((CH, DIM), F32),
            pltpu.VMEM_SHARED((N_PAD, DIM), F32),
            [pltpu.SemaphoreType.DMA] * 4,
            [pltpu.SemaphoreType.DMA] * 4,
            pltpu.SemaphoreType.DMA,
            pltpu.SemaphoreType.DMA,
        ],
    )
    def scat_kernel(u_hbm, src_hbm, dst_hbm, out_hbm, sring, dring, rows0, rows1,
                    acc, ssems, dsems, gsem0, gsem1):
        c = lax.axis_index("c")
        s = lax.axis_index("s")
        w = s * NC + c
        rows = (rows0, rows1)
        gsems = (gsem0, gsem1)

        for t in range(4):
            pltpu.async_copy(src_hbm.at[w, t], sring.at[t], ssems[t])
            pltpu.async_copy(dst_hbm.at[w, t], dring.at[t], dsems[t])

        @pl.loop(0, CH)
        def _(r):
            for cc in range(DIM // 16):
                rows0[r, pl.ds(cc * 16, 16)] = jnp.zeros((16,), F32)

        @pl.loop(0, ROWS_PER_TILE // CH)
        def _(t):
            pltpu.sync_copy(rows0, acc.at[pl.ds(s * ROWS_PER_TILE + t * CH, CH)])

        plsc.subcore_barrier()

        pltpu.make_async_copy(src_hbm.at[w, 0], sring.at[0], ssems[0]).wait()
        pltpu.async_copy(u_hbm.at[sring.at[0]], rows0, gsem0)
        pltpu.make_async_copy(src_hbm.at[w, 1], sring.at[1], ssems[1]).wait()
        pltpu.async_copy(u_hbm.at[sring.at[1]], rows1, gsem1)

        @pl.loop(0, k // 4)
        def _(j4):
            j = j4 * 4
            for t in range(4):
                jj = j + t
                rb = rows[t % 2]
                gs = gsems[t % 2]
                t2 = (t + 2) % 4
                pltpu.make_async_copy(dst_hbm.at[w, jj], dring.at[t], dsems[t]).wait()
                pltpu.make_async_copy(u_hbm.at[sring.at[t]], rb, gs).wait()
                pltpu.sync_copy(rb, acc.at[dring.at[t]], add=True)

                @pl.when(jj + 2 < k)
                def _():
                    pltpu.make_async_copy(
                        src_hbm.at[w, jj + 2], sring.at[t2], ssems[t2]
                    ).wait()
                    pltpu.async_copy(u_hbm.at[sring.at[t2]], rb, gs)

                @pl.when(jj + 4 < k)
                def _():
                    pltpu.async_copy(src_hbm.at[w, jj + 4], sring.at[t], ssems[t])
                    pltpu.async_copy(dst_hbm.at[w, jj + 4], dring.at[t], dsems[t])

        plsc.subcore_barrier()
        pltpu.sync_copy(
            acc.at[pl.ds(s * ROWS_PER_TILE, ROWS_PER_TILE)],
            out_hbm.at[c, pl.ds(s * ROWS_PER_TILE, ROWS_PER_TILE)],
        )

    return scat_kernel(u, src_idx, dst_idx)


def _sc_gather_pair(z, lsrc_idx, ldst_idx):
    kl = lsrc_idx.shape[1]
    l_pad = NW * kl * CH

    @functools.partial(
        pl.kernel,
        out_type=[
            jax.ShapeDtypeStruct((l_pad, DIM), F32),
            jax.ShapeDtypeStruct((l_pad, DIM), F32),
        ],
        mesh=_mesh,
        scratch_types=[
            pltpu.VMEM((kl, CH), jnp.int32),
            pltpu.VMEM((kl, CH), jnp.int32),
            pltpu.VMEM((CH, DIM), F32),
            pltpu.VMEM((CH, DIM), F32),
            pltpu.VMEM((CH, DIM), F32),
            pltpu.VMEM((CH, DIM), F32),
            pltpu.SemaphoreType.DMA,
            [pltpu.SemaphoreType.DMA] * 4,
            [pltpu.SemaphoreType.DMA] * 4,
        ],
    )
    def gat_kernel(z_hbm, ls_hbm, ld_hbm, es_hbm, ed_hbm, sidx, didx,
                   rs0, rd0, rs1, rd1, isem, gsems, wsems):
        c = lax.axis_index("c")
        s = lax.axis_index("s")
        w = s * NC + c
        base = w * kl * CH

        di = pltpu.async_copy(ls_hbm.at[w], sidx, isem)
        dj = pltpu.async_copy(ld_hbm.at[w], didx, isem)
        di.wait()
        dj.wait()

        pltpu.async_copy(z_hbm.at[sidx.at[0]], rs0, gsems[0])
        pltpu.async_copy(z_hbm.at[didx.at[0]], rd0, gsems[1])
        pltpu.async_copy(z_hbm.at[sidx.at[1]], rs1, gsems[2])
        pltpu.async_copy(z_hbm.at[didx.at[1]], rd1, gsems[3])

        @pl.loop(0, kl // 2)
        def _(j2):
            j = j2 * 2

            pltpu.make_async_copy(z_hbm.at[sidx.at[j]], rs0, gsems[0]).wait()
            pltpu.async_copy(rs0, es_hbm.at[pl.ds(base + j * CH, CH)], wsems[0])
            pltpu.make_async_copy(z_hbm.at[didx.at[j]], rd0, gsems[1]).wait()
            pltpu.async_copy(rd0, ed_hbm.at[pl.ds(base + j * CH, CH)], wsems[1])

            pltpu.make_async_copy(z_hbm.at[sidx.at[j + 1]], rs1, gsems[2]).wait()
            pltpu.async_copy(rs1, es_hbm.at[pl.ds(base + (j + 1) * CH, CH)], wsems[2])
            pltpu.make_async_copy(z_hbm.at[didx.at[j + 1]], rd1, gsems[3]).wait()
            pltpu.async_copy(rd1, ed_hbm.at[pl.ds(base + (j + 1) * CH, CH)], wsems[3])

            pltpu.make_async_copy(rs0, es_hbm.at[pl.ds(base + j * CH, CH)], wsems[0]).wait()
            pltpu.make_async_copy(rd0, ed_hbm.at[pl.ds(base + j * CH, CH)], wsems[1]).wait()

            @pl.when(j + 2 < kl)
            def _():
                pltpu.async_copy(z_hbm.at[sidx.at[j + 2]], rs0, gsems[0])
                pltpu.async_copy(z_hbm.at[didx.at[j + 2]], rd0, gsems[1])

            pltpu.make_async_copy(rs1, es_hbm.at[pl.ds(base + (j + 1) * CH, CH)], wsems[2]).wait()
            pltpu.make_async_copy(rd1, ed_hbm.at[pl.ds(base + (j + 1) * CH, CH)], wsems[3]).wait()

            @pl.when(j + 3 < kl)
            def _():
                pltpu.async_copy(z_hbm.at[sidx.at[j + 3]], rs1, gsems[2])
                pltpu.async_copy(z_hbm.at[didx.at[j + 3]], rd1, gsems[3])

    return gat_kernel(z, lsrc_idx, ldst_idx)


def _tc_mm1(x_pad, w1, d0, d1):
    grid = (N_PAD // BLK,)

    def body(x_ref, w_ref, d0_ref, d1_ref, u_ref, dinv_ref):
        deg = d0_ref[...] + d1_ref[...] + 1.0
        dinv = lax.rsqrt(deg)
        dinv_ref[...] = dinv
        h = _dot_bf16(x_ref[...], w_ref[...])
        u_ref[...] = h * dinv[:, None]

    return pl.pallas_call(
        body,
        grid=grid,
        in_specs=[
            pl.BlockSpec((BLK, DIM), lambda i: (i, 0)),
            pl.BlockSpec((DIM, DIM), lambda i: (0, 0)),
            pl.BlockSpec((BLK,), lambda i: (i,)),
            pl.BlockSpec((BLK,), lambda i: (i,)),
        ],
        out_specs=[
            pl.BlockSpec((BLK, DIM), lambda i: (i, 0)),
            pl.BlockSpec((BLK,), lambda i: (i,)),
        ],
        out_shape=[
            jax.ShapeDtypeStruct((N_PAD, DIM), F32),
            jax.ShapeDtypeStruct((N_PAD,), F32),
        ],
    )(x_pad, w1, d0, d1)


def _tc_mm2(sp, u1, dinv, b1, w2):
    grid = (N_PAD // BLK,)

    def body(s0_ref, s1_ref, u_ref, dinv_ref, b_ref, w_ref, o_ref):
        dinv = dinv_ref[...]
        h = dinv[:, None] * (s0_ref[0] + s1_ref[0] + u_ref[...])
        h = jnp.maximum(h + b_ref[...][None, :], 0.0)
        o_ref[...] = _dot_bf16(h, w_ref[...]) * dinv[:, None]

    return pl.pallas_call(
        body,
        grid=grid,
        in_specs=[
            pl.BlockSpec((1, BLK, DIM), lambda i: (0, i, 0)),
            pl.BlockSpec((1, BLK, DIM), lambda i: (1, i, 0)),
            pl.BlockSpec((BLK, DIM), lambda i: (i, 0)),
            pl.BlockSpec((BLK,), lambda i: (i,)),
            pl.BlockSpec((DIM,), lambda i: (0,)),
            pl.BlockSpec((DIM, DIM), lambda i: (0, 0)),
        ],
        out_specs=pl.BlockSpec((BLK, DIM), lambda i: (i, 0)),
        out_shape=jax.ShapeDtypeStruct((N_PAD, DIM), F32),
    )(sp, sp, u1, dinv, b1, w2)


def _tc_z(sp, u2, dinv, b2):
    grid = (N_PAD // BLK,)

    def body(s0_ref, s1_ref, u_ref, dinv_ref, b_ref, o_ref):
        dinv = dinv_ref[...]
        o_ref[...] = (
            dinv[:, None] * (s0_ref[0] + s1_ref[0] + u_ref[...])
            + b_ref[...][None, :]
        )

    return pl.pallas_call(
        body,
        grid=grid,
        in_specs=[
            pl.BlockSpec((1, BLK, DIM), lambda i: (0, i, 0)),
            pl.BlockSpec((1, BLK, DIM), lambda i: (1, i, 0)),
            pl.BlockSpec((BLK, DIM), lambda i: (i, 0)),
            pl.BlockSpec((BLK,), lambda i: (i,)),
            pl.BlockSpec((DIM,), lambda i: (0,)),
        ],
        out_specs=pl.BlockSpec((BLK, DIM), lambda i: (i, 0)),
        out_shape=jax.ShapeDtypeStruct((N_PAD, DIM), F32),
    )(sp, sp, u2, dinv, b2)


def _tc_mlp(es, ed, wp1, bp1, wp2, bp2, wp3pad, bp3):
    l_pad = es.shape[0]
    grid = (l_pad // LBLK,)

    def body(es_ref, ed_ref, w1_ref, b1_ref, w2_ref, b2_ref, w3_ref, b3_ref, o_ref):
        e = es_ref[...] * ed_ref[...]
        a = jnp.maximum(_dot_bf16(e, w1_ref[...]) + b1_ref[...][None, :], 0.0)
        a = jnp.maximum(_dot_bf16(a, w2_ref[...]) + b2_ref[...][None, :], 0.0)
        o_ref[...] = jnp.sum(_dot_bf16(a, w3_ref[...]), axis=1) + jnp.sum(b3_ref[...])

    return pl.pallas_call(
        body,
        grid=grid,
        in_specs=[
            pl.BlockSpec((LBLK, DIM), lambda i: (i, 0)),
            pl.BlockSpec((LBLK, DIM), lambda i: (i, 0)),
            pl.BlockSpec((DIM, DIM), lambda i: (0, 0)),
            pl.BlockSpec((DIM,), lambda i: (0,)),
            pl.BlockSpec((DIM, DIM), lambda i: (0, 0)),
            pl.BlockSpec((DIM,), lambda i: (0,)),
            pl.BlockSpec((DIM, 8), lambda i: (0, 0)),
            pl.BlockSpec((1,), lambda i: (0,)),
        ],
        out_specs=pl.BlockSpec((LBLK,), lambda i: (i,)),
        out_shape=jax.ShapeDtypeStruct((l_pad,), F32),
    )(es, ed, wp1, bp1, wp2, bp2, wp3pad, bp3)


def kernel(x, edge_index, edge_label_index, W1, b1, W2, b2, Wp1, bp1, Wp2, bp2, Wp3, bp3):
    n_trash = N_PAD - N_NODES
    x_pad = jnp.pad(x, ((0, n_trash), (0, 0)))

    e = edge_index.shape[1]
    pad_e = NW * CH * 4 * (-(-e // (NW * CH * 4))) - e
    ar_e = jnp.arange(max(pad_e, 1), dtype=jnp.int32)
    src_c = _pad_chunk(edge_index[0], ar_e % N_NODES, mult=4)
    dst_c = _pad_chunk(edge_index[1], N_NODES + ar_e % n_trash, mult=4)

    l = edge_label_index.shape[1]
    pad_l = NW * CH * 4 * (-(-l // (NW * CH * 4))) - l
    ar_l = jnp.arange(max(pad_l, 1), dtype=jnp.int32)
    lsrc_c = _pad_chunk(edge_label_index[0], ar_l % N_NODES, mult=4)
    ldst_c = _pad_chunk(edge_label_index[1], (ar_l + 7) % N_NODES, mult=4)

    degp = _sc_degree(dst_c)
    u1, dinv = _tc_mm1(x_pad, W1, degp[0], degp[1])
    sp1 = _sc_scatter(u1, src_c, dst_c)
    u2 = _tc_mm2(sp1, u1, dinv, b1, W2)
    sp2 = _sc_scatter(u2, src_c, dst_c)
    z = _tc_z(sp2, u2, dinv, b2)
    wp3pad = jnp.pad(Wp3, ((0, 0), (0, 7)))

    kl = lsrc_c.shape[1]
    half = kl // 2
    parts = []
    for h in range(2):
        sl = slice(h * half, (h + 1) * half)
        es, ed = _sc_gather_pair(z, lsrc_c[:, sl], ldst_c[:, sl])
        parts.append(_tc_mlp(es, ed, Wp1, bp1, Wp2, bp2, wp3pad, bp3))
    a = parts[0].reshape(NW, half * CH)
    b = parts[1].reshape(NW, half * CH)
    scores = jnp.concatenate([a, b], axis=1).reshape(NW * kl * CH)
    return scores[:l]
